# baseline (speedup 1.0000x reference)
import functools

import jax
import jax.numpy as jnp
from jax import lax
from jax.experimental import pallas as pl
from jax.experimental.pallas import tpu as pltpu
from jax.experimental.pallas import tpu_sc as plsc

E = 16
D_STATE = 768
D_ACTION = 64
HIDDEN = 256
N_TOKENS = 16384
BLK = 256
NB = N_TOKENS // BLK + E
P = NB * BLK
D_ACT_PAD = 128
D_XP = 512


def _routing_metadata(policy_indices):
    pol = policy_indices.astype(jnp.int32)
    eids = jnp.arange(E, dtype=jnp.int32)
    onehot = (pol[:, None] == eids[None, :]).astype(jnp.int32)
    cum = jnp.cumsum(onehot, axis=0)
    counts = cum[-1]
    padded = ((counts + BLK - 1) // BLK) * BLK
    pad_off = (jnp.cumsum(padded) - padded).astype(jnp.int32)
    within = jnp.sum(onehot * cum, axis=1) - 1
    ppos = (jnp.sum(onehot * pad_off[None, :], axis=1) + within).astype(jnp.int32)
    bstart = jnp.arange(NB, dtype=jnp.int32)[:, None] * BLK
    block_expert = jnp.clip(
        jnp.sum((pad_off[None, :] <= bstart).astype(jnp.int32), axis=1) - 1,
        0, E - 1).astype(jnp.int32)
    return ppos, block_expert


NC = 2
NS = 16
NW = NC * NS


CBLK = 256


def _round_pack(a, b):
    ua = lax.bitcast_convert_type(a, jnp.uint32)
    ub = lax.bitcast_convert_type(b, jnp.uint32)
    hi = (ua + jnp.uint32(0x8000)) & jnp.uint32(0xFFFF0000)
    lo = (ub + jnp.uint32(0x8000)) >> jnp.uint32(16)
    return lax.bitcast_convert_type(hi | lo, jnp.float32)


def _concat_body(lat_ref, act_ref, x_ref):
    i = pl.program_id(0)
    lat = lat_ref[...]
    a = lat[:, :D_XP]
    act = act_ref[pl.ds(i * CBLK, CBLK), :]
    z = jnp.zeros((CBLK, D_XP - (D_STATE - D_XP) - D_ACTION), jnp.float32)
    b = jnp.concatenate([lat[:, D_XP:], act, z], axis=1)
    x_ref[...] = _round_pack(a, b)


def _concat_inputs(latents, actions):
    return pl.pallas_call(
        _concat_body,
        grid=(N_TOKENS // CBLK,),
        in_specs=[
            pl.BlockSpec((CBLK, D_STATE), lambda i: (i, 0)),
            pl.BlockSpec((N_TOKENS, D_ACTION), lambda i: (0, 0)),
        ],
        out_specs=pl.BlockSpec((CBLK, D_XP), lambda i: (i, 0)),
        out_shape=jax.ShapeDtypeStruct((N_TOKENS, D_XP), jnp.float32),
    )(latents, actions)


_CH = 128


def _scatter_body(idx_hbm, tab_hbm, out_hbm, idx_v, row_v, sem):
    wid = lax.axis_index("s") * NC + lax.axis_index("c")
    rows = N_TOKENS // NW
    base = wid * rows
    for c in range(rows // _CH):
        b = base + c * _CH
        pltpu.sync_copy(idx_hbm.at[pl.ds(b, _CH)], idx_v)
        pltpu.sync_copy(tab_hbm.at[pl.ds(b, _CH)], row_v)
        pltpu.async_copy(row_v, out_hbm.at[idx_v], sem).wait()


def _scatter_rows(idx, table, out_rows, width, dtype):
    fn = pl.kernel(
        _scatter_body,
        out_type=jax.ShapeDtypeStruct((out_rows, width), dtype),
        mesh=plsc.VectorSubcoreMesh(core_axis_name="c", subcore_axis_name="s"),
        scratch_types=[
            pltpu.VMEM((_CH,), jnp.int32),
            pltpu.VMEM((_CH, width), dtype),
            pltpu.SemaphoreType.DMA,
        ],
    )
    return fn(idx, table)


def _gather_body(idx_hbm, tab_hbm, out_hbm, idx_v, row_v, sem):
    wid = lax.axis_index("s") * NC + lax.axis_index("c")
    rows = N_TOKENS // NW
    base = wid * rows
    for c in range(rows // _CH):
        b = base + c * _CH
        pltpu.sync_copy(idx_hbm.at[pl.ds(b, _CH)], idx_v)
        pltpu.async_copy(tab_hbm.at[idx_v], row_v, sem).wait()
        pltpu.sync_copy(row_v, out_hbm.at[pl.ds(b, _CH)])


def _gather_rows(idx, table, width, dtype):
    fn = pl.kernel(
        _gather_body,
        out_type=jax.ShapeDtypeStruct((N_TOKENS, width), dtype),
        mesh=plsc.VectorSubcoreMesh(core_axis_name="c", subcore_axis_name="s"),
        scratch_types=[
            pltpu.VMEM((_CH,), jnp.int32),
            pltpu.VMEM((_CH, width), dtype),
            pltpu.SemaphoreType.DMA,
        ],
    )
    return fn(idx, table)


def _mlp_body(be_ref, x_ref, w1_ref, w1a_ref, b1_ref, w2_ref, b2_ref, out_ref,
              w1b_ref, w1ab_ref, w2b_ref):
    i = pl.program_id(0)

    @pl.when(i == 0)
    def _cast_weights():
        w1b_ref[...] = w1_ref[...].astype(jnp.bfloat16)
        w1ab_ref[...] = w1a_ref[...].astype(jnp.bfloat16)
        w2b_ref[...] = w2_ref[...].astype(jnp.bfloat16)

    e = be_ref[i]
    u = lax.bitcast_convert_type(x_ref[...], jnp.uint32)
    a = lax.bitcast_convert_type(u & jnp.uint32(0xFFFF0000), jnp.float32
                                 ).astype(jnp.bfloat16)
    b = lax.bitcast_convert_type(u << jnp.uint32(16), jnp.float32
                                 ).astype(jnp.bfloat16)
    h = jnp.dot(a, w1b_ref[e, :D_XP, :], preferred_element_type=jnp.float32)
    h = h + jnp.dot(b[:, :D_STATE - D_XP], w1b_ref[e, D_XP:, :],
                    preferred_element_type=jnp.float32)
    h = h + jnp.dot(b[:, D_STATE - D_XP:D_STATE - D_XP + D_ACT_PAD], w1ab_ref[e],
                    preferred_element_type=jnp.float32)
    h = jnp.maximum(h + b1_ref[e, 0], 0.0).astype(jnp.bfloat16)
    out_ref[...] = jnp.dot(h, w2b_ref[e], preferred_element_type=jnp.float32) + b2_ref[e, 0]


def _grouped_mlp(block_expert, x_s, W1, W1a, b1, W2, b2, interpret=False):
    grid_spec = pltpu.PrefetchScalarGridSpec(
        num_scalar_prefetch=1,
        grid=(NB,),
        in_specs=[
            pl.BlockSpec((BLK, D_XP), lambda i, be: (i, 0)),
            pl.BlockSpec((E, D_STATE, HIDDEN), lambda i, be: (0, 0, 0)),
            pl.BlockSpec((E, D_ACT_PAD, HIDDEN), lambda i, be: (0, 0, 0)),
            pl.BlockSpec((E, 1, HIDDEN), lambda i, be: (0, 0, 0)),
            pl.BlockSpec((E, HIDDEN, D_STATE), lambda i, be: (0, 0, 0)),
            pl.BlockSpec((E, 1, D_STATE), lambda i, be: (0, 0, 0)),
        ],
        out_specs=pl.BlockSpec((BLK, D_STATE), lambda i, be: (i, 0)),
        scratch_shapes=[
            pltpu.VMEM((E, D_STATE, HIDDEN), jnp.bfloat16),
            pltpu.VMEM((E, D_ACT_PAD, HIDDEN), jnp.bfloat16),
            pltpu.VMEM((E, HIDDEN, D_STATE), jnp.bfloat16),
        ],
    )
    return pl.pallas_call(
        _mlp_body,
        grid_spec=grid_spec,
        out_shape=jax.ShapeDtypeStruct((P, D_STATE), jnp.float32),
        compiler_params=pltpu.CompilerParams(
            dimension_semantics=("arbitrary",),
        ),
        interpret=interpret,
    )(block_expert, x_s, W1, W1a, b1, W2, b2)


def kernel(latents, policy_indices, actions, W1, b1, W2, b2):
    ppos, block_expert = _routing_metadata(policy_indices)
    xcat = _concat_inputs(latents, actions)
    x_s = _scatter_rows(ppos, xcat, P, D_XP, jnp.float32)
    W1m = W1[:, :D_STATE, :]
    W1a = jnp.pad(W1[:, D_STATE:, :], ((0, 0), (0, D_ACT_PAD - D_ACTION), (0, 0)))
    out_s = _grouped_mlp(block_expert, x_s, W1m, W1a,
                         b1.reshape(E, 1, HIDDEN), W2,
                         b2.reshape(E, 1, D_STATE))
    return _gather_rows(ppos, out_s, D_STATE, jnp.float32)


# speedup vs baseline: 1.0928x; 1.0928x over previous
import functools

import jax
import jax.numpy as jnp
from jax import lax
from jax.experimental import pallas as pl
from jax.experimental.pallas import tpu as pltpu
from jax.experimental.pallas import tpu_sc as plsc

E = 16
D_STATE = 768
D_ACTION = 64
HIDDEN = 256
N_TOKENS = 16384
BLK = 256
NB = N_TOKENS // BLK + E
P = NB * BLK
D_ACT_PAD = 128
D_XP = 512


def _routing_metadata(policy_indices):
    pol = policy_indices.astype(jnp.int32)
    eids = jnp.arange(E, dtype=jnp.int32)
    onehot = (pol[:, None] == eids[None, :]).astype(jnp.int32)
    cum = jnp.cumsum(onehot, axis=0)
    counts = cum[-1]
    padded = ((counts + BLK - 1) // BLK) * BLK
    pad_off = (jnp.cumsum(padded) - padded).astype(jnp.int32)
    within = jnp.sum(onehot * cum, axis=1) - 1
    ppos = (jnp.sum(onehot * pad_off[None, :], axis=1) + within).astype(jnp.int32)
    bstart = jnp.arange(NB, dtype=jnp.int32)[:, None] * BLK
    block_expert = jnp.clip(
        jnp.sum((pad_off[None, :] <= bstart).astype(jnp.int32), axis=1) - 1,
        0, E - 1).astype(jnp.int32)
    return ppos, block_expert


NC = 2
NS = 16
NW = NC * NS


CBLK = 512


def _round_pack(a, b):
    ua = lax.bitcast_convert_type(a, jnp.uint32)
    ub = lax.bitcast_convert_type(b, jnp.uint32)
    hi = (ua + jnp.uint32(0x8000)) & jnp.uint32(0xFFFF0000)
    lo = (ub + jnp.uint32(0x8000)) >> jnp.uint32(16)
    return lax.bitcast_convert_type(hi | lo, jnp.float32)


def _concat_body(lat_ref, act_ref, x_ref):
    i = pl.program_id(0)
    lat = lat_ref[...]
    a = lat[:, :D_XP]
    act = act_ref[pl.ds(i * CBLK, CBLK), :]
    z = jnp.zeros((CBLK, D_XP - (D_STATE - D_XP) - D_ACTION), jnp.float32)
    b = jnp.concatenate([lat[:, D_XP:], act, z], axis=1)
    x_ref[...] = _round_pack(a, b)


def _concat_inputs(latents, actions):
    return pl.pallas_call(
        _concat_body,
        grid=(N_TOKENS // CBLK,),
        in_specs=[
            pl.BlockSpec((CBLK, D_STATE), lambda i: (i, 0)),
            pl.BlockSpec((N_TOKENS, D_ACTION), lambda i: (0, 0)),
        ],
        out_specs=pl.BlockSpec((CBLK, D_XP), lambda i: (i, 0)),
        out_shape=jax.ShapeDtypeStruct((N_TOKENS, D_XP), jnp.float32),
    )(latents, actions)


_CH = 128


def _scatter_body(idx_hbm, tab_hbm, out_hbm, idx_v, row_v, sem):
    wid = lax.axis_index("s") * NC + lax.axis_index("c")
    rows = N_TOKENS // NW
    base = wid * rows
    for c in range(rows // _CH):
        b = base + c * _CH
        pltpu.sync_copy(idx_hbm.at[pl.ds(b, _CH)], idx_v)
        pltpu.sync_copy(tab_hbm.at[pl.ds(b, _CH)], row_v)
        pltpu.async_copy(row_v, out_hbm.at[idx_v], sem).wait()


def _scatter_rows(idx, table, out_rows, width, dtype):
    fn = pl.kernel(
        _scatter_body,
        out_type=jax.ShapeDtypeStruct((out_rows, width), dtype),
        mesh=plsc.VectorSubcoreMesh(core_axis_name="c", subcore_axis_name="s"),
        scratch_types=[
            pltpu.VMEM((_CH,), jnp.int32),
            pltpu.VMEM((_CH, width), dtype),
            pltpu.SemaphoreType.DMA,
        ],
    )
    return fn(idx, table)


def _gather_body(idx_hbm, tab_hbm, out_hbm, idx_v, row_v, sem):
    wid = lax.axis_index("s") * NC + lax.axis_index("c")
    rows = N_TOKENS // NW
    base = wid * rows
    for c in range(rows // _CH):
        b = base + c * _CH
        pltpu.sync_copy(idx_hbm.at[pl.ds(b, _CH)], idx_v)
        pltpu.async_copy(tab_hbm.at[idx_v], row_v, sem).wait()
        pltpu.sync_copy(row_v, out_hbm.at[pl.ds(b, _CH)])


def _gather_rows(idx, table, width, dtype):
    fn = pl.kernel(
        _gather_body,
        out_type=jax.ShapeDtypeStruct((N_TOKENS, width), dtype),
        mesh=plsc.VectorSubcoreMesh(core_axis_name="c", subcore_axis_name="s"),
        scratch_types=[
            pltpu.VMEM((_CH,), jnp.int32),
            pltpu.VMEM((_CH, width), dtype),
            pltpu.SemaphoreType.DMA,
        ],
    )
    return fn(idx, table)


def _mlp_body(be_ref, x_ref, w1_ref, b1_ref, w2_ref, b2_ref, out_ref,
              w1b_ref, w1ab_ref, w2b_ref):
    i = pl.program_id(0)

    @pl.when(i == 0)
    def _cast_weights():
        w1b_ref[...] = w1_ref[:, :D_STATE, :].astype(jnp.bfloat16)
        w1ab_ref[...] = jnp.concatenate(
            [w1_ref[:, D_STATE:, :],
             jnp.zeros((E, D_ACT_PAD - D_ACTION, HIDDEN), jnp.float32)],
            axis=1).astype(jnp.bfloat16)
        w2b_ref[...] = w2_ref[...].astype(jnp.bfloat16)

    e = be_ref[i]
    u = lax.bitcast_convert_type(x_ref[...], jnp.uint32)
    a = lax.bitcast_convert_type(u & jnp.uint32(0xFFFF0000), jnp.float32
                                 ).astype(jnp.bfloat16)
    b = lax.bitcast_convert_type(u << jnp.uint32(16), jnp.float32
                                 ).astype(jnp.bfloat16)
    h = jnp.dot(a, w1b_ref[e, :D_XP, :], preferred_element_type=jnp.float32)
    h = h + jnp.dot(b[:, :D_STATE - D_XP], w1b_ref[e, D_XP:, :],
                    preferred_element_type=jnp.float32)
    h = h + jnp.dot(b[:, D_STATE - D_XP:D_STATE - D_XP + D_ACT_PAD], w1ab_ref[e],
                    preferred_element_type=jnp.float32)
    h = jnp.maximum(h + b1_ref[e, 0], 0.0).astype(jnp.bfloat16)
    out_ref[...] = jnp.dot(h, w2b_ref[e], preferred_element_type=jnp.float32) + b2_ref[e, 0]


def _grouped_mlp(block_expert, x_s, W1, b1, W2, b2, interpret=False):
    grid_spec = pltpu.PrefetchScalarGridSpec(
        num_scalar_prefetch=1,
        grid=(NB,),
        in_specs=[
            pl.BlockSpec((BLK, D_XP), lambda i, be: (i, 0)),
            pl.BlockSpec((E, D_STATE + D_ACTION, HIDDEN), lambda i, be: (0, 0, 0)),
            pl.BlockSpec((E, 1, HIDDEN), lambda i, be: (0, 0, 0)),
            pl.BlockSpec((E, HIDDEN, D_STATE), lambda i, be: (0, 0, 0)),
            pl.BlockSpec((E, 1, D_STATE), lambda i, be: (0, 0, 0)),
        ],
        out_specs=pl.BlockSpec((BLK, D_STATE), lambda i, be: (i, 0)),
        scratch_shapes=[
            pltpu.VMEM((E, D_STATE, HIDDEN), jnp.bfloat16),
            pltpu.VMEM((E, D_ACT_PAD, HIDDEN), jnp.bfloat16),
            pltpu.VMEM((E, HIDDEN, D_STATE), jnp.bfloat16),
        ],
    )
    return pl.pallas_call(
        _mlp_body,
        grid_spec=grid_spec,
        out_shape=jax.ShapeDtypeStruct((P, D_STATE), jnp.float32),
        compiler_params=pltpu.CompilerParams(
            dimension_semantics=("arbitrary",),
        ),
        interpret=interpret,
    )(block_expert, x_s, W1, b1, W2, b2)


def kernel(latents, policy_indices, actions, W1, b1, W2, b2):
    ppos, block_expert = _routing_metadata(policy_indices)
    xcat = _concat_inputs(latents, actions)
    x_s = _scatter_rows(ppos, xcat, P, D_XP, jnp.float32)
    out_s = _grouped_mlp(block_expert, x_s, W1,
                         b1.reshape(E, 1, HIDDEN), W2,
                         b2.reshape(E, 1, D_STATE))
    return _gather_rows(ppos, out_s, D_STATE, jnp.float32)
